# baseline (speedup 1.0000x reference)
import functools

import jax
import jax.numpy as jnp
from jax import lax
from jax.experimental import pallas as pl

_N = 10000
_D = 128
_EPS = 1e-5
_TILE = 1000
_GRID = _N // _TILE


def _stats_body(a_ref, mu_ref, c_ref):
    i = pl.program_id(0)
    a = a_ref[...]
    mu = jnp.sum(a, axis=1)
    c = jnp.stack([
        lax.dot_general(a[k], a[k], (((0,), (0,)), ((), ())),
                        preferred_element_type=jnp.float32,
                        precision=lax.Precision.HIGHEST)
        for k in range(4)
    ])

    @pl.when(i == 0)
    def _():
        mu_ref[...] = mu
        c_ref[...] = c

    @pl.when(i > 0)
    def _():
        mu_ref[...] = mu_ref[...] + mu
        c_ref[...] = c_ref[...] + c


def _stats_call(a4):
    return pl.pallas_call(
        _stats_body,
        grid=(_GRID,),
        in_specs=[pl.BlockSpec((4, _TILE, _D), lambda i: (0, i, 0))],
        out_specs=[pl.BlockSpec((4, _D), lambda i: (0, 0)),
                   pl.BlockSpec((4, _D, _D), lambda i: (0, 0, 0))],
        out_shape=[jax.ShapeDtypeStruct((4, _D), jnp.float32),
                   jax.ShapeDtypeStruct((4, _D, _D), jnp.float32)],
    )(a4)


def _affine_body(nm, mu_ref, c_ref, w_ref, g_ref, be_ref, a_ref, d_ref):
    for m in range(nm):
        for k in range(4):
            wmk = w_ref[m, k]
            mu = mu_ref[k][None, :] * (1.0 / _N)
            t = jnp.dot(mu, wmk, preferred_element_type=jnp.float32,
                        precision=lax.Precision.HIGHEST)
            cw = jnp.dot(c_ref[k], wmk, preferred_element_type=jnp.float32,
                         precision=lax.Precision.HIGHEST)
            q = jnp.sum(wmk * cw, axis=0)[None, :] * (1.0 / _N)
            var = jnp.maximum(q - t * t, 0.0)
            a = g_ref[m, k][None, :] * lax.rsqrt(var + _EPS)
            a_ref[m, k] = a[0]
            d_ref[m, k] = be_ref[m, k] - (a * t)[0]


def _affine_call(nm, mu, c, w, g, be):
    return pl.pallas_call(
        functools.partial(_affine_body, nm),
        out_shape=[jax.ShapeDtypeStruct((nm, 4, _D), jnp.float32),
                   jax.ShapeDtypeStruct((nm, 4, _D), jnp.float32)],
    )(mu, c, w, g, be)


def _apply_body(nm, a4_ref, w_ref, asc_ref, dsh_ref, wmix_ref, out_ref):
    for m in range(nm):
        acc = jnp.zeros((_TILE, _D), jnp.float32)
        for k in range(4):
            nh = jnp.dot(a4_ref[k], w_ref[m, k],
                         preferred_element_type=jnp.float32,
                         precision=lax.Precision.HIGHEST)
            y = nh * asc_ref[m, k][None, :] + dsh_ref[m, k][None, :]
            acc = acc + wmix_ref[m, k] * jnp.maximum(y, 0.0)
        out_ref[m] = acc


def _apply_call(nm, a4, w, asc, dsh, wmix):
    return pl.pallas_call(
        functools.partial(_apply_body, nm),
        grid=(_GRID,),
        in_specs=[
            pl.BlockSpec((4, _TILE, _D), lambda i: (0, i, 0)),
            pl.BlockSpec((nm, 4, _D, _D), lambda i: (0, 0, 0, 0)),
            pl.BlockSpec((nm, 4, _D), lambda i: (0, 0, 0)),
            pl.BlockSpec((nm, 4, _D), lambda i: (0, 0, 0)),
            pl.BlockSpec((nm, 4), lambda i: (0, 0)),
        ],
        out_specs=[pl.BlockSpec((nm, _TILE, _D), lambda i: (0, i, 0))],
        out_shape=[jax.ShapeDtypeStruct((nm, _N, _D), jnp.float32)],
    )(a4, w, asc, dsh, wmix)


def _aggregates(h, src, dst, cnt):
    s = jax.ops.segment_sum(h[src], dst, num_segments=_N)
    mx = jax.ops.segment_max(h[src], dst, num_segments=_N)
    mx = jnp.where((cnt > 0)[:, None], mx, 0.0)
    mean = s / jnp.maximum(cnt, 1.0)[:, None]
    return jnp.stack([s, mean, mx, h])


def _stage(nm, a4, w, g, be, wmix):
    mu, c = _stats_call(a4)
    asc, dsh = _affine_call(nm, mu, c, w, g, be)
    (out,) = _apply_call(nm, a4, w, asc, dsh, wmix)
    return out


def kernel(x, h_in, edge_index, weights, W, b, gamma, beta):
    src, dst = edge_index[0], edge_index[1]
    cnt = jax.ops.segment_sum(jnp.ones((src.shape[0],), jnp.float32), dst,
                              num_segments=_N)

    a4 = _aggregates(x, src, dst, cnt)
    out01 = _stage(2, a4, W[0:2], gamma[0:2], beta[0:2], weights[0:2])
    s1 = out01[0]

    a4b = _aggregates(s1, src, dst, cnt)
    out2 = _stage(1, a4b, W[2:3], gamma[2:3], beta[2:3], weights[2:3])

    s2 = out01[1] + out2[0]
    return jnp.stack([s1, s2])


# speedup vs baseline: 1.1611x; 1.1611x over previous
import functools

import jax
import jax.numpy as jnp
from jax import lax
from jax.experimental import pallas as pl
from jax.experimental.pallas import tpu as pltpu
from jax.experimental.pallas import tpu_sc as plsc

_N = 10000
_E = 320000
_D = 128
_DP = 128
_NP = 10240
_EPS = 1e-5
_TILE = 1000
_GRID = _N // _TILE

_NTILES = 32
_EPT = _E // _NTILES
_G = 400
_NCHUNK = _EPT // _G
_RPT = _NP // 16


def _stats_body(a_ref, mu_ref, c_ref):
    i = pl.program_id(0)
    a = a_ref[...]
    mu = jnp.sum(a, axis=1)
    c = jnp.stack([
        lax.dot_general(a[k], a[k], (((0,), (0,)), ((), ())),
                        preferred_element_type=jnp.float32,
                        precision=lax.Precision.HIGHEST)
        for k in range(4)
    ])

    @pl.when(i == 0)
    def _():
        mu_ref[...] = mu
        c_ref[...] = c

    @pl.when(i > 0)
    def _():
        mu_ref[...] = mu_ref[...] + mu
        c_ref[...] = c_ref[...] + c


def _stats_call(a4):
    return pl.pallas_call(
        _stats_body,
        grid=(_GRID,),
        in_specs=[pl.BlockSpec((4, _TILE, _D), lambda i: (0, i, 0))],
        out_specs=[pl.BlockSpec((4, _D), lambda i: (0, 0)),
                   pl.BlockSpec((4, _D, _D), lambda i: (0, 0, 0))],
        out_shape=[jax.ShapeDtypeStruct((4, _D), jnp.float32),
                   jax.ShapeDtypeStruct((4, _D, _D), jnp.float32)],
    )(a4)


def _affine_body(nm, mu_ref, c_ref, w_ref, g_ref, be_ref, a_ref, d_ref):
    for m in range(nm):
        for k in range(4):
            wmk = w_ref[m, k]
            mu = mu_ref[k][None, :] * (1.0 / _N)
            t = jnp.dot(mu, wmk, preferred_element_type=jnp.float32,
                        precision=lax.Precision.HIGHEST)
            cw = jnp.dot(c_ref[k], wmk, preferred_element_type=jnp.float32,
                         precision=lax.Precision.HIGHEST)
            q = jnp.sum(wmk * cw, axis=0)[None, :] * (1.0 / _N)
            var = jnp.maximum(q - t * t, 0.0)
            a = g_ref[m, k][None, :] * lax.rsqrt(var + _EPS)
            a_ref[m, k] = a[0]
            d_ref[m, k] = be_ref[m, k] - (a * t)[0]


def _affine_call(nm, mu, c, w, g, be):
    return pl.pallas_call(
        functools.partial(_affine_body, nm),
        out_shape=[jax.ShapeDtypeStruct((nm, 4, _D), jnp.float32),
                   jax.ShapeDtypeStruct((nm, 4, _D), jnp.float32)],
    )(mu, c, w, g, be)


def _apply_body(nm, a4_ref, w_ref, asc_ref, dsh_ref, wmix_ref, out_ref):
    for m in range(nm):
        acc = jnp.zeros((_TILE, _D), jnp.float32)
        for k in range(4):
            nh = jnp.dot(a4_ref[k], w_ref[m, k],
                         preferred_element_type=jnp.float32,
                         precision=lax.Precision.HIGHEST)
            y = nh * asc_ref[m, k][None, :] + dsh_ref[m, k][None, :]
            acc = acc + wmix_ref[m, k] * jnp.maximum(y, 0.0)
        out_ref[m] = acc


def _apply_call(nm, a4, w, asc, dsh, wmix):
    return pl.pallas_call(
        functools.partial(_apply_body, nm),
        grid=(_GRID,),
        in_specs=[
            pl.BlockSpec((4, _TILE, _D), lambda i: (0, i, 0)),
            pl.BlockSpec((nm, 4, _D, _D), lambda i: (0, 0, 0, 0)),
            pl.BlockSpec((nm, 4, _D), lambda i: (0, 0, 0)),
            pl.BlockSpec((nm, 4, _D), lambda i: (0, 0, 0)),
            pl.BlockSpec((nm, 4), lambda i: (0, 0)),
        ],
        out_specs=[pl.BlockSpec((nm, _TILE, _D), lambda i: (0, i, 0))],
        out_shape=[jax.ShapeDtypeStruct((nm, _N, _D), jnp.float32)],
    )(a4, w, asc, dsh, wmix)


_HALF = 5120
_ACCR = 5248
_EPS_ = None


def _sc_sum_body(h_hbm, src_hbm, dst_hbm, out_hbm, stag, sbuf, dbuf, zbuf, acc,
                 sem):
    c = lax.axis_index("c")
    s = lax.axis_index("s")
    lo = c * _HALF

    z16 = jnp.zeros((16,), jnp.float32)
    for r in range(8):
        for j in range(_DP // 16):
            zbuf[r, pl.ds(j * 16, 16)] = z16

    def zrow(i, carry):
        pltpu.sync_copy(zbuf, acc.at[pl.ds(s * (_ACCR // 16) + 8 * i, 8)])
        return carry

    lax.fori_loop(0, _ACCR // 16 // 8, zrow, 0)
    plsc.subcore_barrier()

    def chunk(i, carry):
        eb = s * (_E // 16) + i * _G
        pltpu.sync_copy(src_hbm.at[pl.ds(eb, _G)], sbuf)
        pltpu.sync_copy(dst_hbm.at[pl.ds(eb, _G)], dbuf)
        pltpu.async_copy(h_hbm.at[sbuf], stag, sem).wait()
        for g in range(_G // 16):
            d = dbuf[pl.ds(g * 16, 16)]
            own = (d >= lo) & (d < lo + _HALF)
            dbuf[pl.ds(g * 16, 16)] = jnp.where(own, d - lo, _HALF)
        pltpu.sync_copy(stag, acc.at[dbuf], add=True)
        return carry

    lax.fori_loop(0, _E // 16 // _G, chunk, 0)
    plsc.subcore_barrier()

    bounce = stag.at[pl.ds(0, 320)]
    rb = s * 320
    pltpu.sync_copy(acc.at[pl.ds(rb, 320)], bounce)
    pltpu.sync_copy(bounce, out_hbm.at[c, pl.ds(rb, 320), :])


def _sc_sum_call(h, src, dst):
    mesh = plsc.VectorSubcoreMesh(core_axis_name="c", subcore_axis_name="s")
    return pl.kernel(
        _sc_sum_body,
        out_type=jax.ShapeDtypeStruct((2, _HALF, _DP), jnp.float32),
        mesh=mesh,
        scratch_types=[
            pltpu.VMEM((_G, _DP), jnp.float32),
            pltpu.VMEM((_G,), jnp.int32),
            pltpu.VMEM((_G,), jnp.int32),
            pltpu.VMEM((8, _DP), jnp.float32),
            pltpu.VMEM_SHARED((_ACCR, _DP), jnp.float32),
            pltpu.SemaphoreType.DMA,
        ],
    )(h, src, dst)


def _aggregates(h, src, dst, cnt):
    s2 = _sc_sum_call(h, src, dst)
    s = jnp.concatenate([s2[0], s2[1, :_N - _HALF]], axis=0)
    mx = jax.ops.segment_max(h[src], dst, num_segments=_N)
    mx = jnp.where((cnt > 0)[:, None], mx, 0.0)
    mean = s / jnp.maximum(cnt, 1.0)[:, None]
    return jnp.stack([s, mean, mx, h])


def _stage(nm, a4, w, g, be, wmix):
    mu, c = _stats_call(a4)
    asc, dsh = _affine_call(nm, mu, c, w, g, be)
    (out,) = _apply_call(nm, a4, w, asc, dsh, wmix)
    return out


def kernel(x, h_in, edge_index, weights, W, b, gamma, beta):
    src, dst = edge_index[0], edge_index[1]
    cnt = jax.ops.segment_sum(jnp.ones((_E,), jnp.float32), dst,
                              num_segments=_N)

    a4 = _aggregates(x, src, dst, cnt)
    out01 = _stage(2, a4, W[0:2], gamma[0:2], beta[0:2], weights[0:2])
    s1 = out01[0]

    a4b = _aggregates(s1, src, dst, cnt)
    out2 = _stage(1, a4b, W[2:3], gamma[2:3], beta[2:3], weights[2:3])

    s2 = out01[1] + out2[0]
    return jnp.stack([s1, s2])


# speedup vs baseline: 1.1703x; 1.0079x over previous
import functools

import jax
import jax.numpy as jnp
from jax import lax
from jax.experimental import pallas as pl
from jax.experimental.pallas import tpu as pltpu
from jax.experimental.pallas import tpu_sc as plsc

_N = 10000
_E = 320000
_D = 128
_DP = 128
_NP = 10240
_EPS = 1e-5
_TILE = 1000
_GRID = _N // _TILE

_NTILES = 32
_EPT = _E // _NTILES
_G = 320
_NCHUNK = _EPT // _G
_RPT = _NP // 16


def _stats_body(a_ref, mu_ref, c_ref):
    i = pl.program_id(0)
    a = a_ref[...]
    mu = jnp.sum(a, axis=1)
    c = jnp.stack([
        lax.dot_general(a[k], a[k], (((0,), (0,)), ((), ())),
                        preferred_element_type=jnp.float32,
                        precision=lax.Precision.HIGHEST)
        for k in range(4)
    ])

    @pl.when(i == 0)
    def _():
        mu_ref[...] = mu
        c_ref[...] = c

    @pl.when(i > 0)
    def _():
        mu_ref[...] = mu_ref[...] + mu
        c_ref[...] = c_ref[...] + c


def _stats_call(a4):
    return pl.pallas_call(
        _stats_body,
        grid=(_GRID,),
        in_specs=[pl.BlockSpec((4, _TILE, _D), lambda i: (0, i, 0))],
        out_specs=[pl.BlockSpec((4, _D), lambda i: (0, 0)),
                   pl.BlockSpec((4, _D, _D), lambda i: (0, 0, 0))],
        out_shape=[jax.ShapeDtypeStruct((4, _D), jnp.float32),
                   jax.ShapeDtypeStruct((4, _D, _D), jnp.float32)],
    )(a4)


def _affine_body(nm, mu_ref, c_ref, w_ref, g_ref, be_ref, a_ref, d_ref):
    for m in range(nm):
        for k in range(4):
            wmk = w_ref[m, k]
            mu = mu_ref[k][None, :] * (1.0 / _N)
            t = jnp.dot(mu, wmk, preferred_element_type=jnp.float32,
                        precision=lax.Precision.HIGHEST)
            cw = jnp.dot(c_ref[k], wmk, preferred_element_type=jnp.float32,
                         precision=lax.Precision.HIGHEST)
            q = jnp.sum(wmk * cw, axis=0)[None, :] * (1.0 / _N)
            var = jnp.maximum(q - t * t, 0.0)
            a = g_ref[m, k][None, :] * lax.rsqrt(var + _EPS)
            a_ref[m, k] = a[0]
            d_ref[m, k] = be_ref[m, k] - (a * t)[0]


def _affine_call(nm, mu, c, w, g, be):
    return pl.pallas_call(
        functools.partial(_affine_body, nm),
        out_shape=[jax.ShapeDtypeStruct((nm, 4, _D), jnp.float32),
                   jax.ShapeDtypeStruct((nm, 4, _D), jnp.float32)],
    )(mu, c, w, g, be)


def _apply_body(nm, a4_ref, w_ref, asc_ref, dsh_ref, wmix_ref, out_ref):
    for m in range(nm):
        acc = jnp.zeros((_TILE, _D), jnp.float32)
        for k in range(4):
            nh = jnp.dot(a4_ref[k], w_ref[m, k],
                         preferred_element_type=jnp.float32,
                         precision=lax.Precision.HIGHEST)
            y = nh * asc_ref[m, k][None, :] + dsh_ref[m, k][None, :]
            acc = acc + wmix_ref[m, k] * jnp.maximum(y, 0.0)
        out_ref[m] = acc


def _apply_call(nm, a4, w, asc, dsh, wmix):
    return pl.pallas_call(
        functools.partial(_apply_body, nm),
        grid=(_GRID,),
        in_specs=[
            pl.BlockSpec((4, _TILE, _D), lambda i: (0, i, 0)),
            pl.BlockSpec((nm, 4, _D, _D), lambda i: (0, 0, 0, 0)),
            pl.BlockSpec((nm, 4, _D), lambda i: (0, 0, 0)),
            pl.BlockSpec((nm, 4, _D), lambda i: (0, 0, 0)),
            pl.BlockSpec((nm, 4), lambda i: (0, 0)),
        ],
        out_specs=[pl.BlockSpec((nm, _TILE, _D), lambda i: (0, i, 0))],
        out_shape=[jax.ShapeDtypeStruct((nm, _N, _D), jnp.float32)],
    )(a4, w, asc, dsh, wmix)


_HALF = 5120
_ACCR = 5248
_EPS_ = None


def _vtake(v, idx):
    dn = lax.GatherDimensionNumbers(offset_dims=(), collapsed_slice_dims=(0,),
                                    start_index_map=(0,))
    return lax.gather(v, idx[:, None], dn, (1,),
                      mode=lax.GatherScatterMode.PROMISE_IN_BOUNDS)


_CAP2 = 16384


def _sc_sum_body(h_hbm, src_hbm, dst_hbm, out_hbm, stag, sbuf, dbuf, zbuf,
                 csrc, cdst, gsrc, gdst, acc, sem):
    c = lax.axis_index("c")
    s = lax.axis_index("s")
    lo = c * _HALF
    iota = lax.iota(jnp.int32, 16)
    fifteen = iota * 0 + 15

    z16 = jnp.zeros((16,), jnp.float32)
    for r in range(8):
        for j in range(_DP // 16):
            zbuf[r, pl.ds(j * 16, 16)] = z16

    def zrow(i, carry):
        pltpu.sync_copy(zbuf, acc.at[pl.ds(s * (_ACCR // 16) + 8 * i, 8)])
        return carry

    lax.fori_loop(0, _ACCR // 16 // 8, zrow, 0)
    plsc.subcore_barrier()

    def pref(i, carry):
        csrc[pl.ds(i * 16, 16)] = jnp.zeros((16,), jnp.int32)
        cdst[pl.ds(i * 16, 16)] = jnp.full((16,), _HALF, jnp.int32)
        return carry

    lax.fori_loop(0, _CAP2 // 16, pref, 0)

    def chunk(i, offv):
        eb = s * (_E // 16) + i * 2000
        pltpu.sync_copy(src_hbm.at[pl.ds(eb, 2000)], sbuf)
        pltpu.sync_copy(dst_hbm.at[pl.ds(eb, 2000)], dbuf)
        for g in range(2000 // 16):
            d = dbuf[pl.ds(g * 16, 16)]
            sv = sbuf[pl.ds(g * 16, 16)]
            dl = d - lo
            own = (dl >= 0) & (dl < _HALF)
            cum = plsc.cumsum(jnp.where(own, 1, 0))
            slot = jnp.clip(offv + cum - 1, 0, _CAP2 - 1)
            plsc.store_scatter(csrc, [slot], sv, mask=own)
            plsc.store_scatter(cdst, [slot], dl, mask=own)
            offv = offv + _vtake(cum, fifteen)
        return offv

    offv = lax.fori_loop(0, _E // 16 // 2000, chunk, iota * 0)

    def batch(p):
        for g in range(_G // 16):
            gsrc[pl.ds(g * 16, 16)] = csrc[pl.ds(p * _G + g * 16, 16)]
            gdst[pl.ds(g * 16, 16)] = cdst[pl.ds(p * _G + g * 16, 16)]
        pltpu.async_copy(h_hbm.at[gsrc], stag, sem).wait()
        pltpu.sync_copy(stag, acc.at[gdst], add=True)
        return p + 1

    lax.while_loop(lambda p: jnp.any((p * _G) < offv), batch, jnp.int32(0))
    plsc.subcore_barrier()

    bounce = stag.at[pl.ds(0, 320)]
    rb = s * 320
    pltpu.sync_copy(acc.at[pl.ds(rb, 320)], bounce)
    pltpu.sync_copy(bounce, out_hbm.at[c, pl.ds(rb, 320), :])


def _sc_sum_call(h, src, dst):
    mesh = plsc.VectorSubcoreMesh(core_axis_name="c", subcore_axis_name="s")
    return pl.kernel(
        _sc_sum_body,
        out_type=jax.ShapeDtypeStruct((2, _HALF, _DP), jnp.float32),
        mesh=mesh,
        compiler_params=pltpu.CompilerParams(needs_layout_passes=False),
        scratch_types=[
            pltpu.VMEM((_G, _DP), jnp.float32),
            pltpu.VMEM((2000,), jnp.int32),
            pltpu.VMEM((2000,), jnp.int32),
            pltpu.VMEM((8, _DP), jnp.float32),
            pltpu.VMEM((_CAP2,), jnp.int32),
            pltpu.VMEM((_CAP2,), jnp.int32),
            pltpu.VMEM((_G,), jnp.int32),
            pltpu.VMEM((_G,), jnp.int32),
            pltpu.VMEM_SHARED((_ACCR, _DP), jnp.float32),
            pltpu.SemaphoreType.DMA,
        ],
    )(h, src, dst)


def _aggregates(h, src, dst, cnt):
    s2 = _sc_sum_call(h, src, dst)
    s = jnp.concatenate([s2[0], s2[1, :_N - _HALF]], axis=0)
    mx = jax.ops.segment_max(h[src], dst, num_segments=_N)
    mx = jnp.where((cnt > 0)[:, None], mx, 0.0)
    mean = s / jnp.maximum(cnt, 1.0)[:, None]
    return jnp.stack([s, mean, mx, h])


def _stage(nm, a4, w, g, be, wmix):
    mu, c = _stats_call(a4)
    asc, dsh = _affine_call(nm, mu, c, w, g, be)
    (out,) = _apply_call(nm, a4, w, asc, dsh, wmix)
    return out


def kernel(x, h_in, edge_index, weights, W, b, gamma, beta):
    src, dst = edge_index[0], edge_index[1]
    cnt = jax.ops.segment_sum(jnp.ones((_E,), jnp.float32), dst,
                              num_segments=_N)

    a4 = _aggregates(x, src, dst, cnt)
    out01 = _stage(2, a4, W[0:2], gamma[0:2], beta[0:2], weights[0:2])
    s1 = out01[0]

    a4b = _aggregates(s1, src, dst, cnt)
    out2 = _stage(1, a4b, W[2:3], gamma[2:3], beta[2:3], weights[2:3])

    s2 = out01[1] + out2[0]
    return jnp.stack([s1, s2])
